# Initial kernel scaffold; baseline (speedup 1.0000x reference)
import jax
import jax.numpy as jnp
from jax.experimental import pallas as pl


def kernel(rois, feature_map, img_metas):
    raise NotImplementedError("write your pallas kernel here")


# speedup vs baseline: 2.5744x; 2.5744x over previous
import jax
import jax.numpy as jnp
from jax import lax
from jax.experimental import pallas as pl
from jax.experimental.pallas import tpu as pltpu
from jax.experimental.pallas import tpu_sc as plsc

NUM_ROIS = 1000
B, H, W, C = 2, 256, 256, 256
POOL = 7
NW = 32
RPW = 32
NROWS = 112
F32 = jnp.float32
I32 = jnp.int32


def _body(table, roiflat, out, roiv, idxb, rows, outv, sem):
    wid = lax.axis_index("s") * 2 + lax.axis_index("c")
    pltpu.sync_copy(roiflat.at[pl.ds(wid * (RPW * 8), RPW * 8)], roiv)
    iota = lax.iota(I32, 16)
    iota_f = iota.astype(F32)
    lmask = iota < 14

    @pl.loop(0, RPW)
    def roi_loop(r):
        roi = wid * RPW + r

        @pl.when(roi < NUM_ROIS)
        def _():
            base = r * 8

            def splat_roi(col):
                return plsc.load_gather(
                    roiv, [jnp.full((16,), base + col, I32)])

            bf = splat_roi(0)
            y1 = splat_roi(1)
            x1 = splat_roi(2)
            y2 = splat_roi(3)
            x2 = splat_roi(4)
            bi = jnp.clip(bf.astype(I32), 0, B - 1)
            hscale = (y2 - y1) * float(H - 1) / 13.0
            wscale = (x2 - x1) * float(W - 1) / 13.0
            in_y = y1 * float(H - 1) + iota_f * hscale
            in_x = x1 * float(W - 1) + iota_f * wscale
            ty = jnp.clip(in_y.astype(I32), 0, H - 2)
            tx = jnp.clip(in_x.astype(I32), 0, W - 2)
            yl = in_y - ty.astype(F32)
            xl = in_x - tx.astype(F32)
            ybt = (bi * H + ty) * W

            def splat_lane(v, k):
                return v.at[jnp.full((16,), k, I32)].get(
                    mode="promise_in_bounds")

            for oy in range(POOL):
                for iy in range(2):
                    ysp = splat_lane(ybt, 2 * oy + iy)
                    rowt = ysp + tx
                    rowb = rowt + W
                    g0 = iy * 4 * 14
                    plsc.store_scatter(idxb, [iota + g0], rowt, mask=lmask)
                    plsc.store_scatter(idxb, [iota + (g0 + 14)], rowt + 1,
                                       mask=lmask)
                    plsc.store_scatter(idxb, [iota + (g0 + 28)], rowb,
                                       mask=lmask)
                    plsc.store_scatter(idxb, [iota + (g0 + 42)], rowb + 1,
                                       mask=lmask)
                pltpu.async_copy(table.at[idxb], rows, sem).wait()
                yl0 = splat_lane(yl, 2 * oy)
                yl1 = splat_lane(yl, 2 * oy + 1)
                for ox in range(POOL):
                    xs0 = splat_lane(xl, 2 * ox)
                    xs1 = splat_lane(xl, 2 * ox + 1)

                    @pl.loop(0, 16)
                    def chan_loop(cq):
                        off = cq * 16

                        def bil(iy, j, xs, ys):
                            b0 = 56 * iy
                            tl = rows[b0 + j, pl.ds(off, 16)]
                            tr = rows[b0 + 14 + j, pl.ds(off, 16)]
                            bl = rows[b0 + 28 + j, pl.ds(off, 16)]
                            br = rows[b0 + 42 + j, pl.ds(off, 16)]
                            top = tl + (tr - tl) * xs
                            bot = bl + (br - bl) * xs
                            return top + (bot - top) * ys

                        v00 = bil(0, 2 * ox, xs0, yl0)
                        v01 = bil(0, 2 * ox + 1, xs1, yl0)
                        v10 = bil(1, 2 * ox, xs0, yl1)
                        v11 = bil(1, 2 * ox + 1, xs1, yl1)
                        outv[oy * POOL + ox, pl.ds(off, 16)] = jnp.maximum(
                            jnp.maximum(v00, v01), jnp.maximum(v10, v11))

            pltpu.sync_copy(outv, out.at[roi])


def _roialign_sc(table, roiflat):
    mesh = plsc.VectorSubcoreMesh(core_axis_name="c", subcore_axis_name="s")
    f = pl.kernel(
        _body,
        out_type=jax.ShapeDtypeStruct((NUM_ROIS, POOL * POOL, C), F32),
        mesh=mesh,
        compiler_params=pltpu.CompilerParams(needs_layout_passes=False),
        scratch_types=[
            pltpu.VMEM((RPW * 8,), F32),
            pltpu.VMEM((NROWS,), I32),
            pltpu.VMEM((NROWS, C), F32),
            pltpu.VMEM((POOL * POOL, C), F32),
            pltpu.SemaphoreType.DMA,
        ],
    )
    return f(table, roiflat)


def kernel(rois, feature_map, img_metas):
    del img_metas
    table = feature_map.reshape(B * H * W, C)
    roiflat = jnp.pad(rois, ((0, NW * RPW - NUM_ROIS), (0, 3))).reshape(-1)
    out = _roialign_sc(table, roiflat)
    return out.reshape(NUM_ROIS, POOL, POOL, C)


# speedup vs baseline: 3.6369x; 1.4127x over previous
import jax
import jax.numpy as jnp
from jax import lax
from jax.experimental import pallas as pl
from jax.experimental.pallas import tpu as pltpu
from jax.experimental.pallas import tpu_sc as plsc

NUM_ROIS = 1000
B, H, W, C = 2, 256, 256, 256
POOL = 7
NW = 32
RPW = 32
NROWS = 112
F32 = jnp.float32
I32 = jnp.int32


def _body(table, roiflat, out, roiv, idx0, idx1, rows0, rows1, outv, sem0,
          sem1):
    wid = lax.axis_index("s") * 2 + lax.axis_index("c")
    pltpu.sync_copy(roiflat.at[pl.ds(wid * (RPW * 8), RPW * 8)], roiv)
    iota = lax.iota(I32, 16)
    iota_f = iota.astype(F32)
    lmask = iota < 14

    @pl.loop(0, RPW)
    def roi_loop(r):
        roi = wid * RPW + r

        @pl.when(roi < NUM_ROIS)
        def _():
            base = r * 8

            def splat_roi(col):
                return plsc.load_gather(
                    roiv, [jnp.full((16,), base + col, I32)])

            bf = splat_roi(0)
            y1 = splat_roi(1)
            x1 = splat_roi(2)
            y2 = splat_roi(3)
            x2 = splat_roi(4)
            bi = jnp.clip(bf.astype(I32), 0, B - 1)
            hscale = (y2 - y1) * float(H - 1) / 13.0
            wscale = (x2 - x1) * float(W - 1) / 13.0
            in_y = y1 * float(H - 1) + iota_f * hscale
            in_x = x1 * float(W - 1) + iota_f * wscale
            ty = jnp.clip(in_y.astype(I32), 0, H - 2)
            tx = jnp.clip(in_x.astype(I32), 0, W - 2)
            yl = in_y - ty.astype(F32)
            xl = in_x - tx.astype(F32)
            ybt = (bi * H + ty) * W

            def splat_lane(v, k):
                return v.at[jnp.full((16,), k, I32)].get(
                    mode="promise_in_bounds")

            bufs = [(idx0, rows0, sem0), (idx1, rows1, sem1)]

            def start_gather(oy):
                idxb, rows, sem = bufs[oy % 2]
                for iy in range(2):
                    ysp = splat_lane(ybt, 2 * oy + iy)
                    rowt = ysp + tx
                    rowb = rowt + W
                    g0 = iy * 4 * 14
                    plsc.store_scatter(idxb, [iota + g0], rowt, mask=lmask)
                    plsc.store_scatter(idxb, [iota + (g0 + 14)], rowt + 1,
                                       mask=lmask)
                    plsc.store_scatter(idxb, [iota + (g0 + 28)], rowb,
                                       mask=lmask)
                    plsc.store_scatter(idxb, [iota + (g0 + 42)], rowb + 1,
                                       mask=lmask)
                return pltpu.async_copy(table.at[idxb], rows, sem)

            dma = start_gather(0)
            for oy in range(POOL):
                rows = bufs[oy % 2][1]
                next_dma = start_gather(oy + 1) if oy + 1 < POOL else None
                dma.wait()
                dma = next_dma
                yl0 = splat_lane(yl, 2 * oy)
                yl1 = splat_lane(yl, 2 * oy + 1)
                for ox in range(POOL):
                    xs0 = splat_lane(xl, 2 * ox)
                    xs1 = splat_lane(xl, 2 * ox + 1)

                    @pl.loop(0, 16)
                    def chan_loop(cq):
                        off = cq * 16

                        def bil(iy, j, xs, ys):
                            b0 = 56 * iy
                            tl = rows[b0 + j, pl.ds(off, 16)]
                            tr = rows[b0 + 14 + j, pl.ds(off, 16)]
                            bl = rows[b0 + 28 + j, pl.ds(off, 16)]
                            br = rows[b0 + 42 + j, pl.ds(off, 16)]
                            top = tl + (tr - tl) * xs
                            bot = bl + (br - bl) * xs
                            return top + (bot - top) * ys

                        v00 = bil(0, 2 * ox, xs0, yl0)
                        v01 = bil(0, 2 * ox + 1, xs1, yl0)
                        v10 = bil(1, 2 * ox, xs0, yl1)
                        v11 = bil(1, 2 * ox + 1, xs1, yl1)
                        outv[oy * POOL + ox, pl.ds(off, 16)] = jnp.maximum(
                            jnp.maximum(v00, v01), jnp.maximum(v10, v11))

            pltpu.sync_copy(outv, out.at[roi])


def _roialign_sc(table, roiflat):
    mesh = plsc.VectorSubcoreMesh(core_axis_name="c", subcore_axis_name="s")
    f = pl.kernel(
        _body,
        out_type=jax.ShapeDtypeStruct((NUM_ROIS, POOL * POOL, C), F32),
        mesh=mesh,
        compiler_params=pltpu.CompilerParams(needs_layout_passes=False),
        scratch_types=[
            pltpu.VMEM((RPW * 8,), F32),
            pltpu.VMEM((NROWS,), I32),
            pltpu.VMEM((NROWS,), I32),
            pltpu.VMEM((NROWS, C), F32),
            pltpu.VMEM((NROWS, C), F32),
            pltpu.VMEM((POOL * POOL, C), F32),
            pltpu.SemaphoreType.DMA,
            pltpu.SemaphoreType.DMA,
        ],
    )
    return f(table, roiflat)


def kernel(rois, feature_map, img_metas):
    del img_metas
    table = feature_map.reshape(B * H * W, C)
    roiflat = jnp.pad(rois, ((0, NW * RPW - NUM_ROIS), (0, 3))).reshape(-1)
    out = _roialign_sc(table, roiflat)
    return out.reshape(NUM_ROIS, POOL, POOL, C)
